# baseline (speedup 1.0000x reference)
import functools

import jax
import jax.numpy as jnp
from jax import lax
from jax.experimental import pallas as pl
from jax.experimental.pallas import tpu as pltpu
from jax.experimental.pallas import tpu_sc as plsc

NUM_CHANNEL = 8
VOCAB = 100000
HIDDEN = 128
MBS = 4
SEQ = 2048

LANES = 16

_info = plsc.get_sparse_core_info()
NC = _info.num_cores
NS = _info.num_subcores
NW = NC * NS
SPW = SEQ // NW
CS = 32
NSK = SPW // CS
NCHUNK = MBS * NSK
GROW = CS * NUM_CHANNEL
IDW = 128
IDROWS = MBS * SEQ * NUM_CHANNEL // IDW
IROW_PB = SEQ * NUM_CHANNEL // IDW
IROW_PW = MBS * SPW * NUM_CHANNEL // IDW
RPC = GROW // IDW
GSPLIT = 4

_mesh = plsc.VectorSubcoreMesh(core_axis_name="c", subcore_axis_name="s")


@functools.partial(
    pl.kernel,
    mesh=_mesh,
    out_type=jax.ShapeDtypeStruct((SEQ * MBS, HIDDEN), jnp.float32),
    scratch_types=[
        pltpu.VMEM((IROW_PW, IDW), jnp.int32),
        pltpu.VMEM((IROW_PW, IDW), jnp.int32),
        pltpu.VMEM((3, GROW, HIDDEN), jnp.float32),
        pltpu.VMEM((2, CS, HIDDEN), jnp.float32),
        pltpu.SemaphoreType.DMA,
        pltpu.SemaphoreType.DMA,
        pltpu.SemaphoreType.DMA,
        pltpu.SemaphoreType.DMA,
        pltpu.SemaphoreType.DMA,
        pltpu.SemaphoreType.DMA,
    ],
)
def _sc_embed(ids_hbm, tab_hbm, out_hbm, ids_raw, ids_v, gbuf, obuf,
              isem, g0, g1, g2, o0, o1):
    wid = lax.axis_index("s") * NC + lax.axis_index("c")
    s0 = wid * SPW
    gsem = (g0, g1, g2)
    osem = (o0, o1)

    rpb = NSK * RPC
    icopies = [
        pltpu.async_copy(
            ids_hbm.at[pl.ds(b * IROW_PB + wid * rpb, rpb)],
            ids_raw.at[pl.ds(b * rpb, rpb)],
            isem,
        )
        for b in range(MBS)
    ]

    pat = (lax.iota(jnp.int32, LANES) & (NUM_CHANNEL - 1)) * VOCAB

    def prep_rows(lo, n):
        def row_body(r, carry):
            for v in range(IDW // LANES):
                sl = pl.ds(v * LANES, LANES)
                ids_v[r, sl] = ids_raw[r, sl] + pat
            return carry
        lax.fori_loop(lo, lo + n, row_body, 0, unroll=False)

    def fire(k, j):
        n = GROW // MBS
        h = n // 2
        return [
            pltpu.async_copy(
                tab_hbm.at[ids_v.at[b * rpb + (k * n) // IDW,
                                    pl.ds((k * n) % IDW + i * h, h)]],
                gbuf.at[j, pl.ds(b * n + i * h, h)],
                gsem[j],
            )
            for b in range(MBS)
            for i in range(2)
        ]

    icopies[0].wait()
    prep_rows(0, rpb)
    gcopies = [fire(0, 0), fire(1, 1), None]
    icopies[1].wait()
    prep_rows(rpb, rpb)
    gcopies[2] = fire(2, 2)
    for b in range(2, MBS):
        icopies[b].wait()
        prep_rows(b * rpb, rpb)

    scopies = [None, None]

    n_pb = GROW // MBS
    for k in range(NCHUNK):
        j = k % 3
        jo = k % 2
        if scopies[jo] is not None:
            scopies[jo].wait()
            scopies[jo] = None
        for cp in gcopies[j]:
            cp.wait()

        def pos_body(p, carry, _j=j, _jo=jo):
            base = (p & (MBS - 1)) * n_pb + (p >> 2) * NUM_CHANNEL
            for h in range(HIDDEN // LANES):
                sl = pl.ds(h * LANES, LANES)
                acc = gbuf[_j, base, sl]
                for c in range(1, NUM_CHANNEL):
                    acc = acc + gbuf[_j, base + c, sl]
                obuf[_jo, p, sl] = acc
            return carry

        lax.fori_loop(0, CS, pos_body, 0, unroll=False)

        if k + 3 < NCHUNK:
            gcopies[j] = fire(k + 3, j)

        scopies[jo] = pltpu.async_copy(
            obuf.at[jo],
            out_hbm.at[pl.ds(wid * SPW * MBS + k * CS, CS)],
            osem[jo],
        )

    for cp in scopies:
        if cp is not None:
            cp.wait()


def kernel(audio_ids, tables):
    ids2 = audio_ids.reshape(IDROWS, IDW)
    flat_tab = tables.reshape(NUM_CHANNEL * VOCAB, HIDDEN)
    out = _sc_embed(ids2, flat_tab)
    return out.reshape(SEQ, MBS, HIDDEN)


# speedup vs baseline: 1.1726x; 1.1726x over previous
import functools

import jax
import jax.numpy as jnp
from jax import lax
from jax.experimental import pallas as pl
from jax.experimental.pallas import tpu as pltpu
from jax.experimental.pallas import tpu_sc as plsc

NUM_CHANNEL = 8
VOCAB = 100000
HIDDEN = 128
MBS = 4
SEQ = 2048

NPOS = SEQ * MBS
LANES = 16

_info = plsc.get_sparse_core_info()
NC = _info.num_cores
NS = _info.num_subcores
NW = NC * NS
PPW = NPOS // NW
CHUNK = 32
NCHUNK = PPW // CHUNK

_mesh = plsc.VectorSubcoreMesh(core_axis_name="c", subcore_axis_name="s")


@functools.partial(
    pl.kernel,
    mesh=_mesh,
    out_type=jax.ShapeDtypeStruct((NPOS, HIDDEN), jnp.float32),
    scratch_types=[
        pltpu.VMEM((NUM_CHANNEL, NCHUNK, CHUNK), jnp.int32),
        pltpu.VMEM((2, NUM_CHANNEL, CHUNK, HIDDEN), jnp.float32),
        pltpu.VMEM((2, CHUNK, HIDDEN), jnp.float32),
        pltpu.SemaphoreType.DMA,
        pltpu.SemaphoreType.DMA,
        pltpu.SemaphoreType.DMA,
        pltpu.SemaphoreType.DMA,
    ],
)
def _sc_embed(ids_hbm, tab_hbm, out_hbm, ids_v, gbuf, obuf, g0, g1, o0, o1):
    wid = lax.axis_index("s") * NC + lax.axis_index("c")
    gsem = (g0, g1)
    osem = (o0, o1)

    for c in range(NUM_CHANNEL):
        pltpu.sync_copy(ids_hbm.at[c, wid], ids_v.at[c])

    def fire(k, j):
        return [
            pltpu.async_copy(
                tab_hbm.at[c].at[ids_v.at[c, k]], gbuf.at[j, c], gsem[j]
            )
            for c in range(NUM_CHANNEL)
        ]

    gcopies = [fire(0, 0), None]
    scopies = [None, None]

    for k in range(NCHUNK):
        j = k % 2
        for cp in gcopies[j]:
            cp.wait()
        if k + 1 < NCHUNK:
            gcopies[(k + 1) % 2] = fire(k + 1, (k + 1) % 2)

        def pos_body(p, carry, _j=j):
            for h in range(HIDDEN // LANES):
                sl = pl.ds(h * LANES, LANES)
                acc = gbuf[_j, 0, p, sl]
                for c in range(1, NUM_CHANNEL):
                    acc = acc + gbuf[_j, c, p, sl]
                obuf[_j, p, sl] = acc
            return carry

        lax.fori_loop(0, CHUNK, pos_body, 0, unroll=False)

        if scopies[j] is not None:
            scopies[j].wait()
        base = wid * PPW + k * CHUNK
        scopies[j] = pltpu.async_copy(
            obuf.at[j], out_hbm.at[pl.ds(base, CHUNK)], osem[j]
        )

    for cp in scopies:
        if cp is not None:
            cp.wait()


def kernel(audio_ids, tables):
    ids_t = jnp.transpose(audio_ids, (2, 1, 0)).reshape(
        NUM_CHANNEL, NW, NCHUNK, CHUNK
    )
    out = _sc_embed(ids_t, tables)
    return out.reshape(SEQ, MBS, HIDDEN)


# speedup vs baseline: 1.3854x; 1.1814x over previous
import functools

import jax
import jax.numpy as jnp
from jax import lax
from jax.experimental import pallas as pl
from jax.experimental.pallas import tpu as pltpu
from jax.experimental.pallas import tpu_sc as plsc

NUM_CHANNEL = 8
VOCAB = 100000
HIDDEN = 128
MBS = 4
SEQ = 2048

NPOS = SEQ * MBS
LANES = 16

_info = plsc.get_sparse_core_info()
NC = _info.num_cores
NS = _info.num_subcores
NW = NC * NS
PPW = NPOS // NW
CHUNK = 32
NCHUNK = PPW // CHUNK
NBUF = 3

_mesh = plsc.VectorSubcoreMesh(core_axis_name="c", subcore_axis_name="s")


@functools.partial(
    pl.kernel,
    mesh=_mesh,
    out_type=jax.ShapeDtypeStruct((NPOS, HIDDEN), jnp.float32),
    scratch_types=[
        pltpu.VMEM((NUM_CHANNEL, NCHUNK, CHUNK), jnp.int32),
        pltpu.VMEM((NBUF, NUM_CHANNEL, CHUNK, HIDDEN), jnp.float32),
        pltpu.VMEM((2, CHUNK, HIDDEN), jnp.float32),
        pltpu.SemaphoreType.DMA,
        pltpu.SemaphoreType.DMA,
        pltpu.SemaphoreType.DMA,
        pltpu.SemaphoreType.DMA,
        pltpu.SemaphoreType.DMA,
        pltpu.SemaphoreType.DMA,
    ],
)
def _sc_embed(ids_hbm, tab_hbm, out_hbm, ids_v, gbuf, obuf,
              isem, g0, g1, g2, o0, o1):
    wid = lax.axis_index("s") * NC + lax.axis_index("c")
    gsem = (g0, g1, g2)
    osem = (o0, o1)

    icopies = [
        pltpu.async_copy(ids_hbm.at[c, wid], ids_v.at[c], isem)
        for c in range(NUM_CHANNEL)
    ]
    for cp in icopies:
        cp.wait()

    def fire(k, j):
        return [
            pltpu.async_copy(
                tab_hbm.at[c].at[ids_v.at[c, k]], gbuf.at[j, c], gsem[j]
            )
            for c in range(NUM_CHANNEL)
        ]

    gcopies = [fire(0, 0), fire(1, 1), None]
    scopies = [None, None]

    for k in range(NCHUNK):
        j = k % NBUF
        jo = k % 2
        for cp in gcopies[j]:
            cp.wait()
        if k + 2 < NCHUNK:
            gcopies[(k + 2) % NBUF] = fire(k + 2, (k + 2) % NBUF)
        if scopies[jo] is not None:
            scopies[jo].wait()
            scopies[jo] = None

        def pos_body(p, carry, _j=j, _jo=jo):
            for h in range(HIDDEN // LANES):
                sl = pl.ds(h * LANES, LANES)
                acc = gbuf[_j, 0, p, sl]
                for c in range(1, NUM_CHANNEL):
                    acc = acc + gbuf[_j, c, p, sl]
                obuf[_jo, p, sl] = acc
            return carry

        lax.fori_loop(0, CHUNK, pos_body, 0, unroll=False)

        base = wid * PPW + k * CHUNK
        scopies[jo] = pltpu.async_copy(
            obuf.at[jo], out_hbm.at[pl.ds(base, CHUNK)], osem[jo]
        )

    for cp in scopies:
        if cp is not None:
            cp.wait()


def kernel(audio_ids, tables):
    ids_t = jnp.transpose(audio_ids, (2, 1, 0)).reshape(
        NUM_CHANNEL, NW, NCHUNK, CHUNK
    )
    out = _sc_embed(ids_t, tables)
    return out.reshape(SEQ, MBS, HIDDEN)


# speedup vs baseline: 1.4050x; 1.0141x over previous
import functools

import jax
import jax.numpy as jnp
from jax import lax
from jax.experimental import pallas as pl
from jax.experimental.pallas import tpu as pltpu
from jax.experimental.pallas import tpu_sc as plsc

NUM_CHANNEL = 8
VOCAB = 100000
HIDDEN = 128
MBS = 4
SEQ = 2048

NPOS = SEQ * MBS
LANES = 16

_info = plsc.get_sparse_core_info()
NC = _info.num_cores
NS = _info.num_subcores
NW = NC * NS
PPW = NPOS // NW
CHUNK = 32
NCHUNK = PPW // CHUNK
NBUF = 3

_mesh = plsc.VectorSubcoreMesh(core_axis_name="c", subcore_axis_name="s")


@functools.partial(
    pl.kernel,
    mesh=_mesh,
    out_type=jax.ShapeDtypeStruct((NPOS, HIDDEN), jnp.float32),
    scratch_types=[
        pltpu.VMEM((NUM_CHANNEL, PPW // 128, 128), jnp.int32),
        pltpu.VMEM((NBUF, NUM_CHANNEL, CHUNK, HIDDEN), jnp.float32),
        pltpu.VMEM((2, CHUNK, HIDDEN), jnp.float32),
        pltpu.SemaphoreType.DMA,
        pltpu.SemaphoreType.DMA,
        pltpu.SemaphoreType.DMA,
        pltpu.SemaphoreType.DMA,
        pltpu.SemaphoreType.DMA,
        pltpu.SemaphoreType.DMA,
    ],
)
def _sc_embed(ids_hbm, tab_hbm, out_hbm, ids_v, gbuf, obuf,
              isem, g0, g1, g2, o0, o1):
    wid = lax.axis_index("s") * NC + lax.axis_index("c")
    gsem = (g0, g1, g2)
    osem = (o0, o1)

    icopies = [
        pltpu.async_copy(
            ids_hbm.at[c, pl.ds(wid * (PPW // 128), PPW // 128)],
            ids_v.at[c], isem,
        )
        for c in range(NUM_CHANNEL)
    ]
    for cp in icopies:
        cp.wait()

    def fire(k, j):
        return [
            pltpu.async_copy(
                tab_hbm.at[c].at[
                    ids_v.at[k * CHUNK // 128, pl.ds((k * CHUNK) % 128, CHUNK)]
                    if False else
                    ids_v.at[c, (k * CHUNK) // 128, pl.ds((k * CHUNK) % 128, CHUNK)]
                ],
                gbuf.at[j, c], gsem[j]
            )
            for c in range(NUM_CHANNEL)
        ]

    gcopies = [fire(0, 0), fire(1, 1), None]
    scopies = [None, None]

    for k in range(NCHUNK):
        j = k % NBUF
        jo = k % 2
        for cp in gcopies[j]:
            cp.wait()
        if k + 2 < NCHUNK:
            gcopies[(k + 2) % NBUF] = fire(k + 2, (k + 2) % NBUF)
        if scopies[jo] is not None:
            scopies[jo].wait()
            scopies[jo] = None

        def pos_body(p, carry, _j=j, _jo=jo):
            for h in range(HIDDEN // LANES):
                sl = pl.ds(h * LANES, LANES)
                acc = gbuf[_j, 0, p, sl]
                for c in range(1, NUM_CHANNEL):
                    acc = acc + gbuf[_j, c, p, sl]
                obuf[_jo, p, sl] = acc
            return carry

        lax.fori_loop(0, CHUNK, pos_body, 0, unroll=False)

        base = wid * PPW + k * CHUNK
        scopies[jo] = pltpu.async_copy(
            obuf.at[jo], out_hbm.at[pl.ds(base, CHUNK)], osem[jo]
        )

    for cp in scopies:
        if cp is not None:
            cp.wait()


def kernel(audio_ids, tables):
    ids_t = jnp.transpose(audio_ids, (2, 1, 0)).reshape(
        NUM_CHANNEL, NPOS // 128, 128
    )
    out = _sc_embed(ids_t, tables)
    return out.reshape(SEQ, MBS, HIDDEN)
